# baseline (speedup 1.0000x reference)
import functools

import jax
import jax.numpy as jnp
from jax import lax
from jax.experimental import pallas as pl
from jax.experimental.pallas import tpu as pltpu
from jax.experimental.pallas import tpu_sc as plsc

D = 256
HALF = D // 2
V_USED = 100000
LANES = 16
NC, NS = 2, 16
NW = NC * NS
ROW_BLK = 2000
CHUNK = 64
NBUF = 2


def _transform_body(emb_ref, w_ref, b_ref, out_ref):
    y = (
        jnp.dot(emb_ref[...], w_ref[...], preferred_element_type=jnp.float32)
        + b_ref[...]
    )
    yb = y.astype(jnp.bfloat16).astype(jnp.float32)
    lo = lax.bitcast_convert_type(yb[:, :HALF], jnp.uint32) >> 16
    hi = lax.bitcast_convert_type(yb[:, HALF:], jnp.uint32) & jnp.uint32(0xFFFF0000)
    out_ref[...] = lax.bitcast_convert_type(lo | hi, jnp.int32)


def _transform_table(emb, w1t, b2d):
    return pl.pallas_call(
        _transform_body,
        grid=(V_USED // ROW_BLK,),
        in_specs=[
            pl.BlockSpec((ROW_BLK, D), lambda i: (i, 0)),
            pl.BlockSpec((D, D), lambda i: (0, 0)),
            pl.BlockSpec((1, D), lambda i: (0, 0)),
        ],
        out_specs=pl.BlockSpec((ROW_BLK, HALF), lambda i: (i, 0)),
        out_shape=jax.ShapeDtypeStruct((V_USED, HALF), jnp.int32),
    )(emb, w1t, b2d)


@functools.lru_cache(maxsize=None)
def _make_sc_gather(n_tok):
    per_w = n_tok // NW
    n_chunks = per_w // CHUNK
    assert n_chunks % NBUF == 0

    def body(tok_hbm, int_hbm, table_hbm, wl_hbm, out_hbm,
             idx_v, int_v, wl_v, *bufs):
        rows = bufs[0:NBUF]
        outs = bufs[NBUF:2 * NBUF]
        gsem = bufs[2 * NBUF:3 * NBUF]
        wsem = bufs[3 * NBUF:4 * NBUF]
        wid = lax.axis_index("s") * NC + lax.axis_index("c")
        base_w = wid * per_w
        pltpu.sync_copy(wl_hbm, wl_v)
        pltpu.sync_copy(tok_hbm.at[pl.ds(base_w, per_w)], idx_v)
        pltpu.sync_copy(int_hbm.at[pl.ds(base_w, per_w)], int_v)

        def start_gather(k, b):
            pltpu.async_copy(
                table_hbm.at[idx_v.at[pl.ds(k * CHUNK, CHUNK)]], rows[b], gsem[b])

        def wait_gather(b):
            pltpu.make_async_copy(
                table_hbm.at[idx_v.at[pl.ds(0, CHUNK)]], rows[b], gsem[b]).wait()

        def wait_write(b):
            pltpu.make_async_copy(
                outs[b], out_hbm.at[pl.ds(base_w, CHUNK)], wsem[b]).wait()

        for b in range(NBUF):
            start_gather(b, b)

        wls0 = tuple(wl_v[pl.ds(i * LANES, LANES)] for i in range(D // LANES))

        def loop_body(j, wls):
            for b in range(NBUF):
                k = j * NBUF + b
                wait_gather(b)

                def group_body(g, wls):
                    iv = int_v[pl.ds(k * CHUNK + g * LANES, LANES)]
                    for t16 in range(LANES):
                        t = g * LANES + t16
                        s = iv[t16]
                        for gg in range(HALF // LANES):
                            packed = rows[b][t, pl.ds(gg * LANES, LANES)]
                            lov = plsc.bitcast(packed << 16, jnp.float32)
                            hiv = plsc.bitcast(
                                packed & jnp.int32(-65536), jnp.float32)
                            outs[b][t, pl.ds(gg * LANES, LANES)] = (
                                lov + s * wls[gg])
                            outs[b][t, pl.ds(HALF + gg * LANES, LANES)] = (
                                hiv + s * wls[HALF // LANES + gg])
                    return wls

                wls = lax.fori_loop(0, CHUNK // LANES, group_body, wls)
                pltpu.async_copy(
                    outs[b], out_hbm.at[pl.ds(base_w + k * CHUNK, CHUNK)], wsem[b])

                @pl.when(k + NBUF < n_chunks)
                def _():
                    wait_write(b)
                    start_gather(k + NBUF, b)
            return wls

        lax.fori_loop(0, n_chunks // NBUF, loop_body, wls0)
        for b in range(NBUF):
            wait_write(b)

    return pl.kernel(
        body,
        out_type=jax.ShapeDtypeStruct((n_tok, D), jnp.float32),
        mesh=plsc.VectorSubcoreMesh(core_axis_name="c", subcore_axis_name="s"),
        compiler_params=pltpu.CompilerParams(needs_layout_passes=False),
        scratch_types=[
            pltpu.VMEM((per_w,), jnp.int32),
            pltpu.VMEM((per_w,), jnp.float32),
            pltpu.VMEM((D,), jnp.float32),
        ]
        + [pltpu.VMEM((CHUNK, HALF), jnp.int32)] * NBUF
        + [pltpu.VMEM((CHUNK, D), jnp.float32)] * NBUF
        + [pltpu.SemaphoreType.DMA] * (2 * NBUF),
    )


def kernel(tokenized_mz, intensities, emb_table, W, b):
    B_, L_ = tokenized_mz.shape
    n_tok = B_ * L_
    w1t = W[:, :D].T
    wl = W[:, D]
    table2 = _transform_table(emb_table, w1t, b.reshape(1, D))
    out = _make_sc_gather(n_tok)(
        tokenized_mz.reshape(n_tok),
        intensities.reshape(n_tok),
        table2,
        wl,
    )
    return out.reshape(B_, L_, D)


# speedup vs baseline: 2.2712x; 2.2712x over previous
import functools

import jax
import jax.numpy as jnp
from jax import lax
from jax.experimental import pallas as pl
from jax.experimental.pallas import tpu as pltpu
from jax.experimental.pallas import tpu_sc as plsc

D = 256
V_USED = 100000
LANES = 16
NC, NS = 2, 16
NW = NC * NS
ROW_BLK = 2000
CHUNK = 128
NBUF = 2


def _transform_body(emb_ref, w_ref, b_ref, out_ref):
    out_ref[...] = (
        jnp.dot(emb_ref[...], w_ref[...], preferred_element_type=jnp.float32)
        + b_ref[...]
    )


def _transform_table(emb, w1t, b2d):
    return pl.pallas_call(
        _transform_body,
        grid=(V_USED // ROW_BLK,),
        in_specs=[
            pl.BlockSpec((ROW_BLK, D), lambda i: (i, 0)),
            pl.BlockSpec((D, D), lambda i: (0, 0)),
            pl.BlockSpec((1, D), lambda i: (0, 0)),
        ],
        out_specs=pl.BlockSpec((ROW_BLK, D), lambda i: (i, 0)),
        out_shape=jax.ShapeDtypeStruct((V_USED, D), jnp.float32),
    )(emb, w1t, b2d)


@functools.lru_cache(maxsize=None)
def _make_sc_gather(n_tok):
    per_w = n_tok // NW
    n_chunks = per_w // CHUNK
    assert n_chunks % NBUF == 0

    def body(tok_hbm, int_hbm, table_hbm, wl_hbm, out_hbm,
             idx_v, int_v, wl_v, *bufs):
        rows = bufs[0:NBUF]
        gsem = bufs[NBUF:2 * NBUF]
        wsem = bufs[2 * NBUF:3 * NBUF]
        wid = lax.axis_index("s") * NC + lax.axis_index("c")
        base_w = wid * per_w
        pltpu.sync_copy(wl_hbm, wl_v)
        pltpu.sync_copy(tok_hbm.at[pl.ds(base_w, per_w)], idx_v)
        pltpu.sync_copy(int_hbm.at[pl.ds(base_w, per_w)], int_v)

        def start_gather(k, b):
            pltpu.async_copy(
                table_hbm.at[idx_v.at[pl.ds(k * CHUNK, CHUNK)]], rows[b], gsem[b])

        def wait_gather(b):
            pltpu.make_async_copy(
                table_hbm.at[idx_v.at[pl.ds(0, CHUNK)]], rows[b], gsem[b]).wait()

        def wait_write(b):
            pltpu.make_async_copy(
                rows[b], out_hbm.at[pl.ds(base_w, CHUNK)], wsem[b]).wait()

        for b in range(NBUF):
            start_gather(b, b)

        wls0 = tuple(wl_v[pl.ds(i * LANES, LANES)] for i in range(D // LANES))

        def loop_body(j, wls):
            for b in range(NBUF):
                k = j * NBUF + b
                wait_gather(b)

                def group_body(g, wls):
                    iv = int_v[pl.ds(k * CHUNK + g * LANES, LANES)]
                    for t16 in range(LANES):
                        t = g * LANES + t16
                        s = iv[t16]
                        for dc in range(D // LANES):
                            sl = pl.ds(dc * LANES, LANES)
                            rows[b][t, sl] = rows[b][t, sl] + s * wls[dc]
                    return wls

                wls = lax.fori_loop(0, CHUNK // LANES, group_body, wls)
                pltpu.async_copy(
                    rows[b], out_hbm.at[pl.ds(base_w + k * CHUNK, CHUNK)], wsem[b])

                @pl.when(k + NBUF < n_chunks)
                def _():
                    wait_write(b)
                    start_gather(k + NBUF, b)
            return wls

        lax.fori_loop(0, n_chunks // NBUF, loop_body, wls0)
        for b in range(NBUF):
            wait_write(b)

    return pl.kernel(
        body,
        out_type=jax.ShapeDtypeStruct((n_tok, D), jnp.float32),
        mesh=plsc.VectorSubcoreMesh(core_axis_name="c", subcore_axis_name="s"),
        compiler_params=pltpu.CompilerParams(needs_layout_passes=False),
        scratch_types=[
            pltpu.VMEM((per_w,), jnp.int32),
            pltpu.VMEM((per_w,), jnp.float32),
            pltpu.VMEM((D,), jnp.float32),
        ]
        + [pltpu.VMEM((CHUNK, D), jnp.float32)] * NBUF
        + [pltpu.SemaphoreType.DMA] * (2 * NBUF),
    )


def kernel(tokenized_mz, intensities, emb_table, W, b):
    B_, L_ = tokenized_mz.shape
    n_tok = B_ * L_
    w1t = W[:, :D].T
    wl = W[:, D]
    table2 = _transform_table(emb_table, w1t, b.reshape(1, D))
    out = _make_sc_gather(n_tok)(
        tokenized_mz.reshape(n_tok),
        intensities.reshape(n_tok),
        table2,
        wl,
    )
    return out.reshape(B_, L_, D)


# speedup vs baseline: 2.5832x; 1.1374x over previous
import functools

import jax
import jax.numpy as jnp
from jax import lax
from jax.experimental import pallas as pl
from jax.experimental.pallas import tpu as pltpu
from jax.experimental.pallas import tpu_sc as plsc

D = 256
V_USED = 100000
LANES = 16
NC, NS = 2, 16
NW = NC * NS
ROW_BLK = 2000
CHUNK = 128
NBUF = 2


HALF = D // 2


def _transform_body(emb_ref, w_ref, b_ref, out_ref):
    y = (
        jnp.dot(emb_ref[...], w_ref[...], preferred_element_type=jnp.float32)
        + b_ref[...]
    )
    yb = y.astype(jnp.bfloat16).astype(jnp.float32)
    lo = lax.bitcast_convert_type(yb[:, :HALF], jnp.uint32) >> 16
    hi = lax.bitcast_convert_type(yb[:, HALF:], jnp.uint32) & jnp.uint32(0xFFFF0000)
    out_ref[...] = lax.bitcast_convert_type(lo | hi, jnp.float32)


def _transform_table(emb, w1t, b2d):
    return pl.pallas_call(
        _transform_body,
        grid=(V_USED // ROW_BLK,),
        in_specs=[
            pl.BlockSpec((ROW_BLK, D), lambda i: (i, 0)),
            pl.BlockSpec((D, D), lambda i: (0, 0)),
            pl.BlockSpec((1, D), lambda i: (0, 0)),
        ],
        out_specs=pl.BlockSpec((ROW_BLK, HALF), lambda i: (i, 0)),
        out_shape=jax.ShapeDtypeStruct((V_USED, HALF), jnp.float32),
    )(emb, w1t, b2d)


@functools.lru_cache(maxsize=None)
def _make_sc_gather(n_tok):
    per_w = n_tok // NW
    n_chunks = per_w // CHUNK
    assert n_chunks % NBUF == 0

    def body(tok_hbm, int_hbm, table_hbm, wl_hbm, out_hbm,
             idx_v, int_v, wl_v, *bufs):
        rows = bufs[0:NBUF]
        gsem = bufs[NBUF:2 * NBUF]
        wsem = bufs[2 * NBUF:3 * NBUF]
        wid = lax.axis_index("s") * NC + lax.axis_index("c")
        base_w = wid * per_w
        pltpu.sync_copy(wl_hbm, wl_v)
        pltpu.sync_copy(tok_hbm.at[pl.ds(base_w, per_w)], idx_v)
        pltpu.sync_copy(int_hbm.at[pl.ds(base_w, per_w)], int_v)

        def start_gather(k, b):
            pltpu.async_copy(
                table_hbm.at[idx_v.at[pl.ds(k * CHUNK, CHUNK)]],
                rows[b].at[:, pl.ds(0, HALF)], gsem[b])

        def wait_gather(b):
            pltpu.make_async_copy(
                table_hbm.at[idx_v.at[pl.ds(0, CHUNK)]],
                rows[b].at[:, pl.ds(0, HALF)], gsem[b]).wait()

        def wait_write(b):
            pltpu.make_async_copy(
                rows[b], out_hbm.at[pl.ds(base_w, CHUNK)], wsem[b]).wait()

        for b in range(NBUF):
            start_gather(b, b)

        wls0 = tuple(wl_v[pl.ds(i * LANES, LANES)] for i in range(D // LANES))

        def loop_body(j, wls):
            for b in range(NBUF):
                k = j * NBUF + b
                wait_gather(b)

                def group_body(g, wls):
                    iv = int_v[pl.ds(k * CHUNK + g * LANES, LANES)]
                    for t16 in range(LANES):
                        t = g * LANES + t16
                        s = iv[t16]
                        for gg in range(HALF // LANES):
                            packed = plsc.bitcast(
                                rows[b][t, pl.ds(gg * LANES, LANES)], jnp.int32)
                            lov = plsc.bitcast(packed << 16, jnp.float32)
                            hiv = plsc.bitcast(
                                packed & jnp.int32(-65536), jnp.float32)
                            rows[b][t, pl.ds(gg * LANES, LANES)] = (
                                lov + s * wls[gg])
                            rows[b][t, pl.ds(HALF + gg * LANES, LANES)] = (
                                hiv + s * wls[HALF // LANES + gg])
                    return wls

                wls = lax.fori_loop(0, CHUNK // LANES, group_body, wls)
                pltpu.async_copy(
                    rows[b], out_hbm.at[pl.ds(base_w + k * CHUNK, CHUNK)], wsem[b])

                @pl.when(k + NBUF < n_chunks)
                def _():
                    wait_write(b)
                    start_gather(k + NBUF, b)
            return wls

        lax.fori_loop(0, n_chunks // NBUF, loop_body, wls0)
        for b in range(NBUF):
            wait_write(b)

    return pl.kernel(
        body,
        out_type=jax.ShapeDtypeStruct((n_tok, D), jnp.float32),
        mesh=plsc.VectorSubcoreMesh(core_axis_name="c", subcore_axis_name="s"),
        compiler_params=pltpu.CompilerParams(needs_layout_passes=False),
        scratch_types=[
            pltpu.VMEM((per_w,), jnp.int32),
            pltpu.VMEM((per_w,), jnp.float32),
            pltpu.VMEM((D,), jnp.float32),
        ]
        + [pltpu.VMEM((CHUNK, D), jnp.float32)] * NBUF
        + [pltpu.SemaphoreType.DMA] * (2 * NBUF),
    )


def kernel(tokenized_mz, intensities, emb_table, W, b):
    B_, L_ = tokenized_mz.shape
    n_tok = B_ * L_
    w1t = W[:, :D].T
    wl = W[:, D]
    table2 = _transform_table(emb_table, w1t, b.reshape(1, D))
    out = _make_sc_gather(n_tok)(
        tokenized_mz.reshape(n_tok),
        intensities.reshape(n_tok),
        table2,
        wl,
    )
    return out.reshape(B_, L_, D)
